# baseline (speedup 1.0000x reference)
import functools

import jax
import jax.numpy as jnp
from jax import lax
from jax.experimental import pallas as pl
from jax.experimental.pallas import tpu as pltpu
from jax.experimental.pallas import tpu_sc as plsc

B = 16384
D = 64
F = 84
NW = 32
BPW = B // NW
H = BPW // 2


def _sc_kernel(yxt_hbm, idx_hbm, table_hbm, out_hbm,
               idx_v, rows_v, yxt_v, g1s, g2s, fs, w1s, w2s, w3s, w4s):
    wid = lax.axis_index("s") * 2 + lax.axis_index("c")
    base = wid * BPW

    pltpu.sync_copy(idx_hbm.at[pl.ds(base, BPW)], idx_v)
    g1 = pltpu.async_copy(table_hbm.at[idx_v.at[pl.ds(0, H)]],
                          rows_v.at[pl.ds(0, H)], g1s)
    g2 = pltpu.async_copy(table_hbm.at[idx_v.at[pl.ds(H, H)]],
                          rows_v.at[pl.ds(H, H)], g2s)

    f = pltpu.async_copy(yxt_hbm.at[pl.ds(base, BPW)], yxt_v, fs)
    f.wait()
    w1 = pltpu.async_copy(yxt_v.at[:, pl.ds(0, 16)],
                          out_hbm.at[pl.ds(base, BPW), 0, pl.ds(0, 16)], w1s)
    w2 = pltpu.async_copy(yxt_v.at[:, pl.ds(16, 4)],
                          out_hbm.at[pl.ds(base, BPW), 0, pl.ds(80, 4)], w2s)

    g1.wait()
    w3 = pltpu.async_copy(rows_v.at[pl.ds(0, H)],
                          out_hbm.at[pl.ds(base, H), 0, pl.ds(16, D)], w3s)
    g2.wait()
    w4 = pltpu.async_copy(rows_v.at[pl.ds(H, H)],
                          out_hbm.at[pl.ds(base + H, H), 0, pl.ds(16, D)], w4s)

    w1.wait(); w2.wait(); w3.wait(); w4.wait()


@jax.jit
def _run(yxt, idx, table):
    mesh = plsc.VectorSubcoreMesh(core_axis_name="c", subcore_axis_name="s")
    f = functools.partial(
        pl.kernel, mesh=mesh,
        compiler_params=pltpu.CompilerParams(use_tc_tiling_on_sc=False),
        out_type=jax.ShapeDtypeStruct((B, 1, F), jnp.float32),
        scratch_types=[
            pltpu.VMEM((BPW,), jnp.int32),
            pltpu.VMEM((BPW, D), jnp.float32),
            pltpu.VMEM((BPW, 20), jnp.float32),
            pltpu.SemaphoreType.DMA,
            pltpu.SemaphoreType.DMA,
            pltpu.SemaphoreType.DMA,
            pltpu.SemaphoreType.DMA,
            pltpu.SemaphoreType.DMA,
            pltpu.SemaphoreType.DMA,
            pltpu.SemaphoreType.DMA,
        ],
    )(_sc_kernel)
    return f(yxt, idx, table)


def kernel(gsp_y_osgb_fourier, gsp_x_osgb_fourier, gsp_id,
           gsp_5_min_time_utc_fourier, emb_table):
    yxt = jnp.concatenate(
        [gsp_y_osgb_fourier[:, 0], gsp_x_osgb_fourier[:, 0],
         gsp_5_min_time_utc_fourier], axis=1)
    idx = gsp_id.astype(jnp.int32)
    return _run(yxt, idx, emb_table)


# speedup vs baseline: 1.2340x; 1.2340x over previous
import functools

import jax
import jax.numpy as jnp
from jax import lax
from jax.experimental import pallas as pl
from jax.experimental.pallas import tpu as pltpu
from jax.experimental.pallas import tpu_sc as plsc

B = 16384
D = 64
W = 128
NW = 32
BPW = B // NW
H = BPW // 2


def _sc_kernel(yxt_hbm, idx_hbm, table_hbm, out_hbm,
               idx_v, rows_v, yxt_v, g1s, g2s, fs, w1s, w2s, w3s, w4s):
    wid = lax.axis_index("s") * 2 + lax.axis_index("c")
    base = wid * BPW

    pltpu.sync_copy(idx_hbm.at[pl.ds(base, BPW)], idx_v)
    g1 = pltpu.async_copy(table_hbm.at[idx_v.at[pl.ds(0, H)]],
                          rows_v.at[pl.ds(0, H)], g1s)
    g2 = pltpu.async_copy(table_hbm.at[idx_v.at[pl.ds(H, H)]],
                          rows_v.at[pl.ds(H, H)], g2s)

    f = pltpu.async_copy(yxt_hbm.at[pl.ds(base, BPW), pl.ds(0, 24)], yxt_v, fs)
    f.wait()
    w1 = pltpu.async_copy(yxt_v.at[:, pl.ds(0, 16)],
                          out_hbm.at[pl.ds(base, BPW), pl.ds(0, 16)], w1s)
    w2 = pltpu.async_copy(yxt_v.at[:, pl.ds(16, 4)],
                          out_hbm.at[pl.ds(base, BPW), pl.ds(80, 4)], w2s)

    g1.wait()
    w3 = pltpu.async_copy(rows_v.at[pl.ds(0, H), pl.ds(0, D)],
                          out_hbm.at[pl.ds(base, H), pl.ds(16, D)], w3s)
    g2.wait()
    w4 = pltpu.async_copy(rows_v.at[pl.ds(H, H), pl.ds(0, D)],
                          out_hbm.at[pl.ds(base + H, H), pl.ds(16, D)], w4s)

    w1.wait(); w2.wait(); w3.wait(); w4.wait()


@jax.jit
def _run(yxt, idx, tablep):
    mesh = plsc.VectorSubcoreMesh(core_axis_name="c", subcore_axis_name="s")
    f = functools.partial(
        pl.kernel, mesh=mesh,
        compiler_params=pltpu.CompilerParams(use_tc_tiling_on_sc=False),
        out_type=jax.ShapeDtypeStruct((B, W), jnp.float32),
        scratch_types=[
            pltpu.VMEM((BPW,), jnp.int32),
            pltpu.VMEM((BPW, W), jnp.float32),
            pltpu.VMEM((BPW, 24), jnp.float32),
            pltpu.SemaphoreType.DMA,
            pltpu.SemaphoreType.DMA,
            pltpu.SemaphoreType.DMA,
            pltpu.SemaphoreType.DMA,
            pltpu.SemaphoreType.DMA,
            pltpu.SemaphoreType.DMA,
            pltpu.SemaphoreType.DMA,
        ],
    )(_sc_kernel)
    return f(yxt, idx, tablep)


def kernel(gsp_y_osgb_fourier, gsp_x_osgb_fourier, gsp_id,
           gsp_5_min_time_utc_fourier, emb_table):
    yxt = jnp.concatenate(
        [gsp_y_osgb_fourier[:, 0], gsp_x_osgb_fourier[:, 0],
         gsp_5_min_time_utc_fourier, jnp.zeros((B, W - 20), jnp.float32)],
        axis=1)
    tablep = jnp.pad(emb_table, ((0, 0), (0, W - D)))
    idx = gsp_id.astype(jnp.int32)
    out = _run(yxt, idx, tablep)
    return out[:, :84][:, None, :]


# speedup vs baseline: 1.3711x; 1.1111x over previous
import functools

import jax
import jax.numpy as jnp
from jax import lax
from jax.experimental import pallas as pl
from jax.experimental.pallas import tpu as pltpu
from jax.experimental.pallas import tpu_sc as plsc

B = 16384
D = 64
W = 128
NW = 32
BPW = B // NW
C = 128
NC = BPW // C


def _sc_kernel(idx_hbm, table_hbm, out_hbm, idx_v, rows_v, gs, ws):
    wid = lax.axis_index("s") * 2 + lax.axis_index("c")
    base = wid * BPW

    pltpu.sync_copy(idx_hbm.at[pl.ds(wid * NC, NC)], idx_v)

    gathers = []
    for j in range(NC):
        gathers.append(pltpu.async_copy(
            table_hbm.at[idx_v.at[j]], rows_v.at[pl.ds(j * C, C)], gs[j]))
    writes = []
    for j in range(NC):
        gathers[j].wait()
        writes.append(pltpu.async_copy(
            rows_v.at[pl.ds(j * C, C)],
            out_hbm.at[pl.ds(base + j * C, C)], ws[j]))
    for wdma in writes:
        wdma.wait()


@jax.jit
def _run(idx2, tablep):
    mesh = plsc.VectorSubcoreMesh(core_axis_name="c", subcore_axis_name="s")
    f = functools.partial(
        pl.kernel, mesh=mesh,
        compiler_params=pltpu.CompilerParams(use_tc_tiling_on_sc=True),
        out_type=jax.ShapeDtypeStruct((B, W), jnp.float32),
        scratch_types=[
            pltpu.VMEM((NC, C), jnp.int32),
            pltpu.VMEM((BPW, W), jnp.float32),
            [pltpu.SemaphoreType.DMA] * NC,
            [pltpu.SemaphoreType.DMA] * NC,
        ],
    )(_sc_kernel)
    return f(idx2, tablep)


def kernel(gsp_y_osgb_fourier, gsp_x_osgb_fourier, gsp_id,
           gsp_5_min_time_utc_fourier, emb_table):
    tablep = jnp.pad(emb_table, ((0, 0), (0, W - D)))
    idx2 = gsp_id.astype(jnp.int32).reshape(B // C, C)
    rows = _run(idx2, tablep)
    out = jnp.concatenate(
        [gsp_y_osgb_fourier[:, 0], gsp_x_osgb_fourier[:, 0],
         rows[:, :D], gsp_5_min_time_utc_fourier], axis=1)
    return out[:, None, :]


# speedup vs baseline: 1.4920x; 1.0881x over previous
import functools

import jax
import jax.numpy as jnp
from jax import lax
from jax.experimental import pallas as pl
from jax.experimental.pallas import tpu as pltpu
from jax.experimental.pallas import tpu_sc as plsc

B = 16384
D = 64
W = 128
NW = 32
BPW = B // NW
C = 128
NC = BPW // C


def _sc_kernel(idx_hbm, table_hbm, out_hbm, idx_v, rows_v, gs, ws):
    wid = lax.axis_index("s") * 2 + lax.axis_index("c")
    base = wid * BPW

    pltpu.sync_copy(idx_hbm.at[pl.ds(wid * NC, NC)], idx_v)

    gathers = []
    for j in range(NC):
        gathers.append(pltpu.async_copy(
            table_hbm.at[idx_v.at[j]], rows_v.at[pl.ds(j * C, C)], gs[j]))
    writes = []
    for j in range(NC):
        gathers[j].wait()
        writes.append(pltpu.async_copy(
            rows_v.at[pl.ds(j * C, C)],
            out_hbm.at[pl.ds(base + j * C, C)], ws[j]))
    for wdma in writes:
        wdma.wait()


@jax.jit
def _run(idx2, tablep):
    mesh = plsc.VectorSubcoreMesh(core_axis_name="c", subcore_axis_name="s")
    f = functools.partial(
        pl.kernel, mesh=mesh,
        compiler_params=pltpu.CompilerParams(use_tc_tiling_on_sc=True),
        out_type=jax.ShapeDtypeStruct((B, W), jnp.float32),
        scratch_types=[
            pltpu.VMEM((NC, C), jnp.int32),
            pltpu.VMEM((BPW, W), jnp.float32),
            [pltpu.SemaphoreType.DMA] * NC,
            [pltpu.SemaphoreType.DMA] * NC,
        ],
    )(_sc_kernel)
    return f(idx2, tablep)


def kernel(gsp_y_osgb_fourier, gsp_x_osgb_fourier, gsp_id,
           gsp_5_min_time_utc_fourier, emb_table):
    eye = jnp.eye(D, W, dtype=jnp.float32)
    tablep = jax.lax.dot(emb_table, eye,
                         precision=jax.lax.Precision.HIGHEST)
    idx2 = gsp_id.astype(jnp.int32).reshape(B // C, C)
    rows = _run(idx2, tablep)
    out = jnp.concatenate(
        [gsp_y_osgb_fourier[:, 0], gsp_x_osgb_fourier[:, 0],
         rows[:, :D], gsp_5_min_time_utc_fourier], axis=1)
    return out[:, None, :]
